# baseline (speedup 1.0000x reference)
import jax
import jax.numpy as jnp
from jax.experimental import pallas as pl

CH = 128
N_USER = 10000
N_ITEM = 10000
N_SEED = 1024


def _pe(t, channels):
    i = jnp.arange(channels // 2)
    div = jnp.exp(-jnp.log(10000.0) * (2.0 * i) / channels)
    ang = t[:, None] * div[None, :]
    pe = jnp.zeros((t.shape[0], channels), dtype=jnp.float32)
    pe = pe.at[:, 0::2].set(jnp.sin(ang))
    pe = pe.at[:, 1::2].set(jnp.cos(ang))
    return pe


def _sage(h_src, h_dst, edge_index, Wl, Wr, b, n_dst):
    src = edge_index[0]
    dst = edge_index[1]
    msg = jnp.take(h_src, src, axis=0)
    summed = jax.ops.segment_sum(msg, dst, num_segments=n_dst)
    cnt = jax.ops.segment_sum(jnp.ones((dst.shape[0],), jnp.float32), dst, num_segments=n_dst)
    mean = summed / jnp.clip(cnt, 1.0)[:, None]
    return mean @ Wl.T + h_dst @ Wr.T + b


def _copy_kernel(a_ref, o_ref):
    o_ref[...] = a_ref[...]


def _pl_copy(x):
    return pl.pallas_call(
        _copy_kernel,
        out_shape=jax.ShapeDtypeStruct(x.shape, x.dtype),
    )(x)


def kernel(x_user, x_item, time_user, time_item, seed_time, edge_index_u2i, edge_index_i2u, batch_user, batch_item, Wt_user, bt_user, Wt_item, bt_item, Wl0_u2i, Wr0_u2i, b0_u2i, Wl0_i2u, Wr0_i2u, b0_i2u, Wl1_u2i, Wr1_u2i, b1_u2i, Wl1_i2u, Wr1_i2u, b1_i2u):
    rel_u = jnp.take(seed_time, batch_user) - time_user
    rel_i = jnp.take(seed_time, batch_item) - time_item
    enc_u = _pe(rel_u, CH) @ Wt_user.T + bt_user
    enc_i = _pe(rel_i, CH) @ Wt_item.T + bt_item
    h_u = x_user + enc_u
    h_i = x_item + enc_i
    new_i = _sage(h_u, h_i, edge_index_u2i, Wl0_u2i, Wr0_u2i, b0_u2i, N_ITEM)
    new_u = _sage(h_i, h_u, edge_index_i2u, Wl0_i2u, Wr0_i2u, b0_i2u, N_USER)
    h_u = jax.nn.relu(new_u)
    h_i = jax.nn.relu(new_i)
    out_i = _sage(h_u, h_i, edge_index_u2i, Wl1_u2i, Wr1_u2i, b1_u2i, N_ITEM)
    out_u = _sage(h_i, h_u, edge_index_i2u, Wl1_i2u, Wr1_i2u, b1_i2u, N_USER)
    return (_pl_copy(out_u), _pl_copy(out_i))


# speedup vs baseline: 4.9253x; 4.9253x over previous
import functools

import jax
import jax.numpy as jnp
from jax import lax
from jax.experimental import pallas as pl
from jax.experimental.pallas import tpu as pltpu
from jax.experimental.pallas import tpu_sc as plsc

CH = 128
N_USER = 10000
N_ITEM = 10000
E = 320000
N_SEED = 1024

N_TILES = 16
EDGES_PER_TILE = E // N_TILES
CHUNK = 80
N_CHUNKS = EDGES_PER_TILE // CHUNK
ROWS_PAD = 10112
ROWS_PER_TILE = ROWS_PAD // N_TILES

BLK = 1000
HIGH = lax.Precision.HIGHEST


def _enc_body(x_ref, t_ref, b_ref, seed_ref, div_ref, we_ref, wo_ref, bt_ref, o_ref):
    b = b_ref[...]
    iot = lax.broadcasted_iota(jnp.int32, (1, N_SEED), 1)
    onehot = b == iot
    rel = jnp.sum(jnp.where(onehot, seed_ref[...], 0.0), axis=1, keepdims=True)
    rel = rel - t_ref[...]
    ang = rel * div_ref[...]
    o_ref[...] = (x_ref[...] + bt_ref[0]
                  + lax.dot(jnp.sin(ang), we_ref[0], precision=HIGH)
                  + lax.dot(jnp.cos(ang), wo_ref[0], precision=HIGH))


def _encode(x_cat, t_cat, b_cat, seed2d, div2d, we, wo, bt):
    n = x_cat.shape[0]
    grid = (n // BLK,)
    half = n // (2 * BLK)
    return pl.pallas_call(
        _enc_body,
        grid=grid,
        in_specs=[
            pl.BlockSpec((BLK, CH), lambda i: (i, 0)),
            pl.BlockSpec((BLK, 1), lambda i: (i, 0)),
            pl.BlockSpec((BLK, 1), lambda i: (i, 0)),
            pl.BlockSpec((1, N_SEED), lambda i: (0, 0)),
            pl.BlockSpec((1, CH // 2), lambda i: (0, 0)),
            pl.BlockSpec((1, CH // 2, CH), lambda i: (i // half, 0, 0)),
            pl.BlockSpec((1, CH // 2, CH), lambda i: (i // half, 0, 0)),
            pl.BlockSpec((1, 1, CH), lambda i: (i // half, 0, 0)),
        ],
        out_specs=pl.BlockSpec((BLK, CH), lambda i: (i, 0)),
        out_shape=jax.ShapeDtypeStruct((n, CH), jnp.float32),
    )(x_cat, t_cat, b_cat, seed2d, div2d, we, wo, bt)


def _combine_body(sum_ref, cnt_ref, h_ref, wl_ref, wr_ref, b_ref, o_ref, *, relu):
    mean = sum_ref[...] / jnp.maximum(cnt_ref[...], 1.0)
    out = (lax.dot(mean, wl_ref[0], precision=HIGH)
           + lax.dot(h_ref[...], wr_ref[0], precision=HIGH)
           + b_ref[0])
    o_ref[...] = jnp.maximum(out, 0.0) if relu else out


def _combine(sum_cat, cnt_cat, h_cat, wl, wr, b, relu):
    n = sum_cat.shape[0]
    grid = (n // BLK,)
    half = n // (2 * BLK)
    return pl.pallas_call(
        functools.partial(_combine_body, relu=relu),
        grid=grid,
        in_specs=[
            pl.BlockSpec((BLK, CH), lambda i: (i, 0)),
            pl.BlockSpec((BLK, 1), lambda i: (i, 0)),
            pl.BlockSpec((BLK, CH), lambda i: (i, 0)),
            pl.BlockSpec((1, CH, CH), lambda i: (i // half, 0, 0)),
            pl.BlockSpec((1, CH, CH), lambda i: (i // half, 0, 0)),
            pl.BlockSpec((1, 1, CH), lambda i: (i // half, 0, 0)),
        ],
        out_specs=pl.BlockSpec((BLK, CH), lambda i: (i, 0)),
        out_shape=jax.ShapeDtypeStruct((n, CH), jnp.float32),
    )(sum_cat, cnt_cat, h_cat, wl, wr, b)


def _agg_body(tabs, src_h, dst_h, zf, out_sum, src_a, dst_a, src_b, dst_b,
              rows0, rows1, acc, gsem0, gsem1, isem0, isem1):
    c = lax.axis_index("c")
    s = lax.axis_index("s")
    r0 = s * ROWS_PER_TILE
    pltpu.sync_copy(zf.at[pl.ds(r0, ROWS_PER_TILE)],
                    acc.at[pl.ds(r0, ROWS_PER_TILE)])
    plsc.subcore_barrier()

    tab = tabs.at[c]
    ebase = c * E + s * EDGES_PER_TILE

    def loop_body(jj, carry):
        off0 = ebase + (2 * jj) * CHUNK
        off1 = off0 + CHUNK
        ia = pltpu.async_copy(src_h.at[pl.ds(off0, CHUNK)], src_a, isem0)
        ib = pltpu.async_copy(dst_h.at[pl.ds(off0, CHUNK)], dst_a, isem0)
        ic = pltpu.async_copy(src_h.at[pl.ds(off1, CHUNK)], src_b, isem1)
        idd = pltpu.async_copy(dst_h.at[pl.ds(off1, CHUNK)], dst_b, isem1)
        ia.wait()
        g0 = pltpu.async_copy(tab.at[src_a], rows0, gsem0)
        ic.wait()
        g1 = pltpu.async_copy(tab.at[src_b], rows1, gsem1)
        g0.wait()
        ib.wait()
        pltpu.sync_copy(rows0, acc.at[dst_a], add=True)
        g1.wait()
        idd.wait()
        pltpu.sync_copy(rows1, acc.at[dst_b], add=True)
        return carry

    lax.fori_loop(0, N_CHUNKS // 2, loop_body, 0)

    plsc.subcore_barrier()
    pltpu.sync_copy(acc.at[pl.ds(r0, ROWS_PER_TILE)],
                    out_sum.at[c, pl.ds(r0, ROWS_PER_TILE)])


_agg = pl.kernel(
    _agg_body,
    out_type=jax.ShapeDtypeStruct((2, ROWS_PAD, CH), jnp.float32),
    mesh=plsc.VectorSubcoreMesh(core_axis_name="c", subcore_axis_name="s"),
    scratch_types=[
        pltpu.VMEM((CHUNK,), jnp.int32),
        pltpu.VMEM((CHUNK,), jnp.int32),
        pltpu.VMEM((CHUNK,), jnp.int32),
        pltpu.VMEM((CHUNK,), jnp.int32),
        pltpu.VMEM((CHUNK, CH), jnp.float32),
        pltpu.VMEM((CHUNK, CH), jnp.float32),
        pltpu.VMEM_SHARED((ROWS_PAD, CH), jnp.float32),
        pltpu.SemaphoreType.DMA,
        pltpu.SemaphoreType.DMA,
        pltpu.SemaphoreType.DMA,
        pltpu.SemaphoreType.DMA,
    ],
)


def kernel(x_user, x_item, time_user, time_item, seed_time, edge_index_u2i,
           edge_index_i2u, batch_user, batch_item, Wt_user, bt_user, Wt_item,
           bt_item, Wl0_u2i, Wr0_u2i, b0_u2i, Wl0_i2u, Wr0_i2u, b0_i2u,
           Wl1_u2i, Wr1_u2i, b1_u2i, Wl1_i2u, Wr1_i2u, b1_i2u):
    ei_u = edge_index_u2i.astype(jnp.int32)
    ei_i = edge_index_i2u.astype(jnp.int32)
    src_all = jnp.concatenate([ei_u[0], ei_i[0]])
    dst_all = jnp.concatenate([ei_u[1], ei_i[1]])
    zf = jnp.zeros((ROWS_PAD, CH), jnp.float32)

    x_cat = jnp.concatenate([x_user, x_item], axis=0)
    t_cat = jnp.concatenate([time_user, time_item]).reshape(-1, 1)
    b_cat = jnp.concatenate([batch_user, batch_item]).astype(jnp.int32).reshape(-1, 1)
    seed2d = seed_time.reshape(1, N_SEED)
    i_half = jnp.arange(CH // 2, dtype=jnp.float32)
    div2d = jnp.exp(-jnp.log(10000.0) * (2.0 * i_half) / CH).reshape(1, CH // 2)
    we = jnp.stack([Wt_user[:, 0::2].T, Wt_item[:, 0::2].T])
    wo = jnp.stack([Wt_user[:, 1::2].T, Wt_item[:, 1::2].T])
    bt = jnp.stack([bt_user.reshape(1, CH), bt_item.reshape(1, CH)])
    h_cat = _encode(x_cat, t_cat, b_cat, seed2d, div2d, we, wo, bt)

    ones_tabs = jnp.ones((2, N_USER, CH), jnp.float32)
    cnts = _agg(ones_tabs, src_all, dst_all, zf)
    cnt_cat = jnp.concatenate([cnts[1, :N_USER, 0:1], cnts[0, :N_ITEM, 0:1]], axis=0)

    sums0 = _agg(h_cat.reshape(2, N_USER, CH), src_all, dst_all, zf)
    sum_cat0 = jnp.concatenate([sums0[1, :N_USER], sums0[0, :N_ITEM]], axis=0)
    wl0 = jnp.stack([Wl0_i2u.T, Wl0_u2i.T])
    wr0 = jnp.stack([Wr0_i2u.T, Wr0_u2i.T])
    b0 = jnp.stack([b0_i2u.reshape(1, CH), b0_u2i.reshape(1, CH)])
    relu_cat = _combine(sum_cat0, cnt_cat, h_cat, wl0, wr0, b0, relu=True)

    sums1 = _agg(relu_cat.reshape(2, N_USER, CH), src_all, dst_all, zf)
    sum_cat1 = jnp.concatenate([sums1[1, :N_USER], sums1[0, :N_ITEM]], axis=0)
    wl1 = jnp.stack([Wl1_i2u.T, Wl1_u2i.T])
    wr1 = jnp.stack([Wr1_i2u.T, Wr1_u2i.T])
    b1 = jnp.stack([b1_i2u.reshape(1, CH), b1_u2i.reshape(1, CH)])
    out_cat = _combine(sum_cat1, cnt_cat, relu_cat, wl1, wr1, b1, relu=False)
    return (out_cat[:N_USER], out_cat[N_USER:])


# speedup vs baseline: 5.7522x; 1.1679x over previous
import functools

import jax
import jax.numpy as jnp
from jax import lax
from jax.experimental import pallas as pl
from jax.experimental.pallas import tpu as pltpu
from jax.experimental.pallas import tpu_sc as plsc

CH = 128
N_USER = 10000
N_ITEM = 10000
E = 320000
N_SEED = 1024

N_TILES = 16
EDGES_PER_TILE = E // N_TILES
CHUNK = 80
N_CHUNKS = EDGES_PER_TILE // CHUNK
ROWS_PAD = 10112
ROWS_PER_TILE = ROWS_PAD // N_TILES

BLK = 1000
HIGH = lax.Precision.HIGHEST


def _enc_body(x_ref, t_ref, b_ref, seed_ref, div_ref, we_ref, wo_ref, bt_ref, o_ref):
    b = b_ref[...]
    iot = lax.broadcasted_iota(jnp.int32, (1, N_SEED), 1)
    onehot = b == iot
    rel = jnp.sum(jnp.where(onehot, seed_ref[...], 0.0), axis=1, keepdims=True)
    rel = rel - t_ref[...]
    ang = rel * div_ref[...]
    o_ref[...] = (x_ref[...] + bt_ref[0]
                  + lax.dot(jnp.sin(ang), we_ref[0], precision=HIGH)
                  + lax.dot(jnp.cos(ang), wo_ref[0], precision=HIGH))


def _encode(x_cat, t_cat, b_cat, seed2d, div2d, we, wo, bt):
    n = x_cat.shape[0]
    grid = (n // BLK,)
    half = n // (2 * BLK)
    return pl.pallas_call(
        _enc_body,
        grid=grid,
        in_specs=[
            pl.BlockSpec((BLK, CH), lambda i: (i, 0)),
            pl.BlockSpec((BLK, 1), lambda i: (i, 0)),
            pl.BlockSpec((BLK, 1), lambda i: (i, 0)),
            pl.BlockSpec((1, N_SEED), lambda i: (0, 0)),
            pl.BlockSpec((1, CH // 2), lambda i: (0, 0)),
            pl.BlockSpec((1, CH // 2, CH), lambda i: (i // half, 0, 0)),
            pl.BlockSpec((1, CH // 2, CH), lambda i: (i // half, 0, 0)),
            pl.BlockSpec((1, 1, CH), lambda i: (i // half, 0, 0)),
        ],
        out_specs=pl.BlockSpec((BLK, CH), lambda i: (i, 0)),
        out_shape=jax.ShapeDtypeStruct((n, CH), jnp.float32),
    )(x_cat, t_cat, b_cat, seed2d, div2d, we, wo, bt)


def _combine_body(sum_ref, cnt_ref, h_ref, wl_ref, wr_ref, b_ref, o_ref, *, relu):
    mean = sum_ref[...] / jnp.maximum(cnt_ref[...], 1.0)
    out = (lax.dot(mean, wl_ref[0], precision=HIGH)
           + lax.dot(h_ref[...], wr_ref[0], precision=HIGH)
           + b_ref[0])
    o_ref[...] = jnp.maximum(out, 0.0) if relu else out


def _combine(sum_cat, cnt_cat, h_cat, wl, wr, b, relu):
    n = sum_cat.shape[0]
    grid = (n // BLK,)
    half = n // (2 * BLK)
    return pl.pallas_call(
        functools.partial(_combine_body, relu=relu),
        grid=grid,
        in_specs=[
            pl.BlockSpec((BLK, CH), lambda i: (i, 0)),
            pl.BlockSpec((BLK, 1), lambda i: (i, 0)),
            pl.BlockSpec((BLK, CH), lambda i: (i, 0)),
            pl.BlockSpec((1, CH, CH), lambda i: (i // half, 0, 0)),
            pl.BlockSpec((1, CH, CH), lambda i: (i // half, 0, 0)),
            pl.BlockSpec((1, 1, CH), lambda i: (i // half, 0, 0)),
        ],
        out_specs=pl.BlockSpec((BLK, CH), lambda i: (i, 0)),
        out_shape=jax.ShapeDtypeStruct((n, CH), jnp.float32),
    )(sum_cat, cnt_cat, h_cat, wl, wr, b)


def _agg_body(tabs, src_h, dst_h, zf, out_sum, src_a, dst_a, src_b, dst_b,
              rows0, rows1, acc, gsem0, gsem1, isem0, isem1):
    c = lax.axis_index("c")
    s = lax.axis_index("s")
    r0 = s * ROWS_PER_TILE
    pltpu.sync_copy(zf.at[pl.ds(r0, ROWS_PER_TILE)],
                    acc.at[pl.ds(r0, ROWS_PER_TILE)])
    plsc.subcore_barrier()

    tab = tabs.at[c]
    ebase = c * E + s * EDGES_PER_TILE

    def loop_body(jj, carry):
        off0 = ebase + (2 * jj) * CHUNK
        off1 = off0 + CHUNK
        ia = pltpu.async_copy(src_h.at[pl.ds(off0, CHUNK)], src_a, isem0)
        ib = pltpu.async_copy(dst_h.at[pl.ds(off0, CHUNK)], dst_a, isem0)
        ic = pltpu.async_copy(src_h.at[pl.ds(off1, CHUNK)], src_b, isem1)
        idd = pltpu.async_copy(dst_h.at[pl.ds(off1, CHUNK)], dst_b, isem1)
        ia.wait()
        g0 = pltpu.async_copy(tab.at[src_a], rows0, gsem0)
        ic.wait()
        g1 = pltpu.async_copy(tab.at[src_b], rows1, gsem1)
        g0.wait()
        ib.wait()
        pltpu.sync_copy(rows0, acc.at[dst_a], add=True)
        g1.wait()
        idd.wait()
        pltpu.sync_copy(rows1, acc.at[dst_b], add=True)
        return carry

    lax.fori_loop(0, N_CHUNKS // 2, loop_body, 0)

    plsc.subcore_barrier()
    pltpu.sync_copy(acc.at[pl.ds(r0, ROWS_PER_TILE)],
                    out_sum.at[c, pl.ds(r0, ROWS_PER_TILE)])


_agg = pl.kernel(
    _agg_body,
    out_type=jax.ShapeDtypeStruct((2, ROWS_PAD, CH), jnp.float32),
    mesh=plsc.VectorSubcoreMesh(core_axis_name="c", subcore_axis_name="s"),
    scratch_types=[
        pltpu.VMEM((CHUNK,), jnp.int32),
        pltpu.VMEM((CHUNK,), jnp.int32),
        pltpu.VMEM((CHUNK,), jnp.int32),
        pltpu.VMEM((CHUNK,), jnp.int32),
        pltpu.VMEM((CHUNK, CH), jnp.float32),
        pltpu.VMEM((CHUNK, CH), jnp.float32),
        pltpu.VMEM_SHARED((ROWS_PAD, CH), jnp.float32),
        pltpu.SemaphoreType.DMA,
        pltpu.SemaphoreType.DMA,
        pltpu.SemaphoreType.DMA,
        pltpu.SemaphoreType.DMA,
    ],
)


def _cnt_body(dst_h, zf, ones_h, out_sum, dst_a, dst_b, rows0, acc,
              isem0, isem1):
    c = lax.axis_index("c")
    s = lax.axis_index("s")
    r0 = s * ROWS_PER_TILE
    pltpu.sync_copy(zf.at[pl.ds(r0, ROWS_PER_TILE)],
                    acc.at[pl.ds(r0, ROWS_PER_TILE)])
    pltpu.sync_copy(ones_h, rows0)
    plsc.subcore_barrier()

    ebase = c * E + s * EDGES_PER_TILE

    def loop_body(jj, carry):
        off0 = ebase + (2 * jj) * CHUNK
        off1 = off0 + CHUNK
        ia = pltpu.async_copy(dst_h.at[pl.ds(off0, CHUNK)], dst_a, isem0)
        ib = pltpu.async_copy(dst_h.at[pl.ds(off1, CHUNK)], dst_b, isem1)
        ia.wait()
        pltpu.sync_copy(rows0, acc.at[dst_a], add=True)
        ib.wait()
        pltpu.sync_copy(rows0, acc.at[dst_b], add=True)
        return carry

    lax.fori_loop(0, N_CHUNKS // 2, loop_body, 0)

    plsc.subcore_barrier()
    pltpu.sync_copy(acc.at[pl.ds(r0, ROWS_PER_TILE)],
                    out_sum.at[c, pl.ds(r0, ROWS_PER_TILE)])


_cnt = pl.kernel(
    _cnt_body,
    out_type=jax.ShapeDtypeStruct((2, ROWS_PAD, CH), jnp.float32),
    mesh=plsc.VectorSubcoreMesh(core_axis_name="c", subcore_axis_name="s"),
    scratch_types=[
        pltpu.VMEM((CHUNK,), jnp.int32),
        pltpu.VMEM((CHUNK,), jnp.int32),
        pltpu.VMEM((CHUNK, CH), jnp.float32),
        pltpu.VMEM_SHARED((ROWS_PAD, CH), jnp.float32),
        pltpu.SemaphoreType.DMA,
        pltpu.SemaphoreType.DMA,
    ],
)


def kernel(x_user, x_item, time_user, time_item, seed_time, edge_index_u2i,
           edge_index_i2u, batch_user, batch_item, Wt_user, bt_user, Wt_item,
           bt_item, Wl0_u2i, Wr0_u2i, b0_u2i, Wl0_i2u, Wr0_i2u, b0_i2u,
           Wl1_u2i, Wr1_u2i, b1_u2i, Wl1_i2u, Wr1_i2u, b1_i2u):
    ei_u = edge_index_u2i.astype(jnp.int32)
    ei_i = edge_index_i2u.astype(jnp.int32)
    src_all = jnp.concatenate([ei_u[0], ei_i[0]])
    dst_all = jnp.concatenate([ei_u[1], ei_i[1]])
    zf = jnp.zeros((ROWS_PAD, CH), jnp.float32)

    x_cat = jnp.concatenate([x_user, x_item], axis=0)
    t_cat = jnp.concatenate([time_user, time_item]).reshape(-1, 1)
    b_cat = jnp.concatenate([batch_user, batch_item]).astype(jnp.int32).reshape(-1, 1)
    seed2d = seed_time.reshape(1, N_SEED)
    i_half = jnp.arange(CH // 2, dtype=jnp.float32)
    div2d = jnp.exp(-jnp.log(10000.0) * (2.0 * i_half) / CH).reshape(1, CH // 2)
    we = jnp.stack([Wt_user[:, 0::2].T, Wt_item[:, 0::2].T])
    wo = jnp.stack([Wt_user[:, 1::2].T, Wt_item[:, 1::2].T])
    bt = jnp.stack([bt_user.reshape(1, CH), bt_item.reshape(1, CH)])
    h_cat = _encode(x_cat, t_cat, b_cat, seed2d, div2d, we, wo, bt)

    ones_h = jnp.ones((CHUNK, CH), jnp.float32)
    cnts = _cnt(dst_all, zf, ones_h)
    cnt_cat = jnp.concatenate([cnts[1, :N_USER, 0:1], cnts[0, :N_ITEM, 0:1]], axis=0)

    sums0 = _agg(h_cat.reshape(2, N_USER, CH), src_all, dst_all, zf)
    sum_cat0 = jnp.concatenate([sums0[1, :N_USER], sums0[0, :N_ITEM]], axis=0)
    wl0 = jnp.stack([Wl0_i2u.T, Wl0_u2i.T])
    wr0 = jnp.stack([Wr0_i2u.T, Wr0_u2i.T])
    b0 = jnp.stack([b0_i2u.reshape(1, CH), b0_u2i.reshape(1, CH)])
    relu_cat = _combine(sum_cat0, cnt_cat, h_cat, wl0, wr0, b0, relu=True)

    sums1 = _agg(relu_cat.reshape(2, N_USER, CH), src_all, dst_all, zf)
    sum_cat1 = jnp.concatenate([sums1[1, :N_USER], sums1[0, :N_ITEM]], axis=0)
    wl1 = jnp.stack([Wl1_i2u.T, Wl1_u2i.T])
    wr1 = jnp.stack([Wr1_i2u.T, Wr1_u2i.T])
    b1 = jnp.stack([b1_i2u.reshape(1, CH), b1_u2i.reshape(1, CH)])
    out_cat = _combine(sum_cat1, cnt_cat, relu_cat, wl1, wr1, b1, relu=False)
    return (out_cat[:N_USER], out_cat[N_USER:])
